# Initial kernel scaffold; baseline (speedup 1.0000x reference)
import jax
import jax.numpy as jnp
from jax.experimental import pallas as pl


def kernel(x, e, edge_index, Wh, We, A, B, C, D, Ew, bn_hg, bn_hb, bn_eg, bn_eb, Wout):
    raise NotImplementedError("write your pallas kernel here")


# speedup vs baseline: 2.0169x; 2.0169x over previous
import functools

import jax
import jax.numpy as jnp
from jax import lax
from jax.experimental import pallas as pl
from jax.experimental.pallas import tpu as pltpu
from jax.experimental.pallas import tpu_sc as plsc

F32 = jnp.float32

H = 256
NLAYER = 4
N = 10000
E = 160000
CH = 64
NCH = 4
TILES = 16
EPT = E // TILES
BE = 80
NBLK = EPT // BE
NPAD = 10112
NPT = NPAD // TILES
NB = 2000
EB = 2000


def _node_body(h_ref, a_ref, b_ref, d_ref, w_ref, uh_ref, dv_ref, et_ref):
    h = h_ref[...]
    uh_ref[...] = jnp.dot(h, a_ref[...], preferred_element_type=F32)
    vh = jnp.dot(h, b_ref[...], preferred_element_type=F32)
    dh = jnp.dot(h, d_ref[...], preferred_element_type=F32)
    ew = jnp.dot(h, w_ref[...], preferred_element_type=F32)
    for c in range(NCH):
        dv_ref[c, :, 0:CH] = dh[:, c * CH:(c + 1) * CH]
        dv_ref[c, :, CH:2 * CH] = vh[:, c * CH:(c + 1) * CH]
    for s in range(2):
        et_ref[s] = ew[:, s * 2 * CH:(s + 1) * 2 * CH]


def _node0_body(x_ref, wh_ref, a_ref, b_ref, d_ref, w_ref,
                h_ref, uh_ref, dv_ref, et_ref):
    h = jnp.dot(x_ref[...], wh_ref[...], preferred_element_type=F32)
    h_ref[...] = h
    uh_ref[...] = jnp.dot(h, a_ref[...], preferred_element_type=F32)
    vh = jnp.dot(h, b_ref[...], preferred_element_type=F32)
    dh = jnp.dot(h, d_ref[...], preferred_element_type=F32)
    ew = jnp.dot(h, w_ref[...], preferred_element_type=F32)
    for c in range(NCH):
        dv_ref[c, :, 0:CH] = dh[:, c * CH:(c + 1) * CH]
        dv_ref[c, :, CH:2 * CH] = vh[:, c * CH:(c + 1) * CH]
    for s in range(2):
        et_ref[s] = ew[:, s * 2 * CH:(s + 1) * 2 * CH]


_W_SPEC = pl.BlockSpec((H, H), lambda i: (0, 0))


def _node_mm(h, a, b, d, w):
    grid = (N // NB,)
    return pl.pallas_call(
        _node_body,
        grid=grid,
        in_specs=[pl.BlockSpec((NB, H), lambda i: (i, 0))] + [_W_SPEC] * 4,
        out_specs=[
            pl.BlockSpec((NB, H), lambda i: (i, 0)),
            pl.BlockSpec((NCH, NB, 2 * CH), lambda i: (0, i, 0)),
            pl.BlockSpec((2, NB, 2 * CH), lambda i: (0, i, 0)),
        ],
        out_shape=[
            jax.ShapeDtypeStruct((N, H), F32),
            jax.ShapeDtypeStruct((NCH, N, 2 * CH), F32),
            jax.ShapeDtypeStruct((2, N, 2 * CH), F32),
        ],
    )(h, a, b, d, w)


def _node0_mm(x, wh, a, b, d, w):
    grid = (N // NB,)
    return pl.pallas_call(
        _node0_body,
        grid=grid,
        in_specs=[pl.BlockSpec((NB, H), lambda i: (i, 0))] + [_W_SPEC] * 5,
        out_specs=[
            pl.BlockSpec((NB, H), lambda i: (i, 0)),
            pl.BlockSpec((NB, H), lambda i: (i, 0)),
            pl.BlockSpec((NCH, NB, 2 * CH), lambda i: (0, i, 0)),
            pl.BlockSpec((2, NB, 2 * CH), lambda i: (0, i, 0)),
        ],
        out_shape=[
            jax.ShapeDtypeStruct((N, H), F32),
            jax.ShapeDtypeStruct((N, H), F32),
            jax.ShapeDtypeStruct((NCH, N, 2 * CH), F32),
            jax.ShapeDtypeStruct((2, N, 2 * CH), F32),
        ],
    )(x, wh, a, b, d, w)


def _edge0_body(e_ref, we_ref, c_ref, ef_ref, ce_ref):
    ef = jnp.dot(e_ref[...], we_ref[...], preferred_element_type=F32)
    ef_ref[...] = ef
    ce = jnp.dot(ef, c_ref[...], preferred_element_type=F32)
    for c in range(NCH):
        ce_ref[c] = ce[:, c * CH:(c + 1) * CH]


def _edge0_mm(e, we, cw):
    grid = (E // EB,)
    de = e.shape[1]
    return pl.pallas_call(
        _edge0_body,
        grid=grid,
        in_specs=[
            pl.BlockSpec((EB, de), lambda i: (i, 0)),
            pl.BlockSpec((de, H), lambda i: (0, 0)),
            _W_SPEC,
        ],
        out_specs=[
            pl.BlockSpec((EB, H), lambda i: (i, 0)),
            pl.BlockSpec((NCH, EB, CH), lambda i: (0, i, 0)),
        ],
        out_shape=[
            jax.ShapeDtypeStruct((E, H), F32),
            jax.ShapeDtypeStruct((NCH, E, CH), F32),
        ],
    )(e, we, cw)


def _edge_body(ef_ref, eh_ref, ae_ref, be_ref, c_ref, efn_ref, ce_ref):
    eh = jnp.concatenate([eh_ref[c] for c in range(NCH)], axis=1)
    ef = ef_ref[...] + jnp.maximum(eh * ae_ref[...] + be_ref[...], 0.0)
    efn_ref[...] = ef
    ce = jnp.dot(ef, c_ref[...], preferred_element_type=F32)
    for c in range(NCH):
        ce_ref[c] = ce[:, c * CH:(c + 1) * CH]


def _edge_mm(ef, ehat, ae, be, cw):
    grid = (E // EB,)
    return pl.pallas_call(
        _edge_body,
        grid=grid,
        in_specs=[
            pl.BlockSpec((EB, H), lambda i: (i, 0)),
            pl.BlockSpec((NCH, EB, CH), lambda i: (0, i, 0)),
            pl.BlockSpec((1, H), lambda i: (0, 0)),
            pl.BlockSpec((1, H), lambda i: (0, 0)),
            _W_SPEC,
        ],
        out_specs=[
            pl.BlockSpec((EB, H), lambda i: (i, 0)),
            pl.BlockSpec((NCH, EB, CH), lambda i: (0, i, 0)),
        ],
        out_shape=[
            jax.ShapeDtypeStruct((E, H), F32),
            jax.ShapeDtypeStruct((NCH, E, CH), F32),
        ],
    )(ef, ehat, ae, be, cw)


def _hhat_body(nd_ref, uh_ref, hh_ref, part_ref):
    num = jnp.concatenate([nd_ref[c, :, 0:CH] for c in range(NCH)], axis=1)
    den = jnp.concatenate([nd_ref[c, :, CH:2 * CH] for c in range(NCH)],
                          axis=1)
    hh = uh_ref[...] + num / (den + 1e-6)
    hh_ref[...] = hh
    part_ref[0, 0] = jnp.sum(hh, axis=0)
    part_ref[0, 1] = jnp.sum(hh * hh, axis=0)


def _hhat(nd, uh):
    grid = (N // NB,)
    return pl.pallas_call(
        _hhat_body,
        grid=grid,
        in_specs=[
            pl.BlockSpec((NCH, NB, 2 * CH), lambda i: (0, i, 0)),
            pl.BlockSpec((NB, H), lambda i: (i, 0)),
        ],
        out_specs=[
            pl.BlockSpec((NB, H), lambda i: (i, 0)),
            pl.BlockSpec((1, 2, H), lambda i: (i, 0, 0)),
        ],
        out_shape=[
            jax.ShapeDtypeStruct((N, H), F32),
            jax.ShapeDtypeStruct((N // NB, 2, H), F32),
        ],
    )(nd, uh)


def _hup_body(hh_ref, h_ref, sa_ref, sb_ref, hn_ref):
    bn = hh_ref[...] * sa_ref[...] + sb_ref[...]
    hn_ref[...] = h_ref[...] + jnp.maximum(bn, 0.0)


def _hup(hh, h, sa, sb):
    grid = (N // NB,)
    return pl.pallas_call(
        _hup_body,
        grid=grid,
        in_specs=[
            pl.BlockSpec((NB, H), lambda i: (i, 0)),
            pl.BlockSpec((NB, H), lambda i: (i, 0)),
            pl.BlockSpec((1, H), lambda i: (0, 0)),
            pl.BlockSpec((1, H), lambda i: (0, 0)),
        ],
        out_specs=pl.BlockSpec((NB, H), lambda i: (i, 0)),
        out_shape=jax.ShapeDtypeStruct((N, H), F32),
    )(hh, h, sa, sb)


def _update(nd, uh, h, g, b):
    hh, part = _hhat(nd, uh)
    m = jnp.sum(part[:, 0, :], axis=0) / N
    ms = jnp.sum(part[:, 1, :], axis=0) / N
    var = ms - m * m
    rstd = 1.0 / jnp.sqrt(var + 1e-5)
    sa = (g * rstd).reshape(1, H)
    sb = (b - m * g * rstd).reshape(1, H)
    return _hup(hh, h, sa, sb)


def _out_body(h_ref, w_ref, o_ref):
    o_ref[...] = jnp.tanh(jnp.dot(h_ref[...], w_ref[...],
                                  preferred_element_type=F32))


def _out_mm(h, wout):
    return pl.pallas_call(
        _out_body,
        in_specs=[
            pl.BlockSpec((N, H), lambda: (0, 0)),
            pl.BlockSpec((H, 8), lambda: (0, 0)),
        ],
        out_specs=pl.BlockSpec((N, 8), lambda: (0, 0)),
        out_shape=jax.ShapeDtypeStruct((N, 8), F32),
    )(h, wout)


def _sc_body(write_ehat, src_hbm, dst_hbm, dv_hbm, et_hbm, ce_hbm, *rest):
    if write_ehat:
        (nd_hbm, ehat_hbm, st_hbm,
         src_v, dst_v, dv_b, et_b, ce_b, sc_b, st_b, acc,
         sem1, sem2) = rest
    else:
        (nd_hbm,
         src_v, dst_v, dv_b, et_b, ce_b, sc_b, st_b, acc,
         sem1, sem2) = rest
    cid = lax.axis_index("c")
    sid = lax.axis_index("s")
    zero = jnp.zeros((16,), F32)

    def zrow(r, _):
        for q in range(8):
            sc_b[r, pl.ds(16 * q, 16)] = zero
        return 0

    for j in range(2):
        c = cid * 2 + j
        lax.fori_loop(0, BE, zrow, 0)
        for k in range(7):
            pltpu.sync_copy(sc_b, acc.at[pl.ds(sid * NPT + k * BE, BE)])
        pltpu.sync_copy(sc_b.at[pl.ds(0, NPT - 7 * BE)],
                        acc.at[pl.ds(sid * NPT + 7 * BE, NPT - 7 * BE)])
        plsc.subcore_barrier()

        def blk(b, carry, c=c, j=j):
            base = sid * EPT + b * BE
            pltpu.sync_copy(src_hbm.at[pl.ds(base, BE)], src_v)
            pltpu.sync_copy(dst_hbm.at[pl.ds(base, BE)], dst_v)
            cp1 = pltpu.async_copy(dv_hbm.at[c].at[src_v], dv_b, sem1)
            cp2 = pltpu.async_copy(et_hbm.at[cid].at[dst_v], et_b, sem2)
            pltpu.sync_copy(ce_hbm.at[c, pl.ds(base, BE)], ce_b)
            cp1.wait()
            cp2.wait()

            def row(r, rc):
                s1n, s2n = [], []
                for q in range(4):
                    sl = pl.ds(16 * q, 16)
                    sl2 = pl.ds(CH + 16 * q, 16)
                    sle = pl.ds(j * CH + 16 * q, 16)
                    ehv = ce_b[r, sl] + dv_b[r, sl] + et_b[r, sle]
                    sg = 1.0 / (1.0 + jnp.exp(-ehv))
                    if write_ehat:
                        ce_b[r, sl] = ehv
                        s1n.append(rc[q] + ehv)
                        s2n.append(rc[4 + q] + ehv * ehv)
                    sc_b[r, sl] = sg * dv_b[r, sl2]
                    sc_b[r, sl2] = sg
                if write_ehat:
                    return tuple(s1n + s2n)
                return rc

            carry = lax.fori_loop(0, BE, row, carry)
            if write_ehat:
                pltpu.sync_copy(ce_b, ehat_hbm.at[c, pl.ds(base, BE)])
            pltpu.sync_copy(sc_b, acc.at[dst_v], add=True)
            return carry

        carry = lax.fori_loop(0, NBLK, blk, (zero,) * 8)
        if write_ehat:
            for q in range(4):
                st_b[q] = carry[q]
                st_b[4 + q] = carry[4 + q]
            pltpu.sync_copy(st_b, st_hbm.at[c, sid])
        plsc.subcore_barrier()
        pltpu.sync_copy(acc.at[pl.ds(sid * NPT, NPT)],
                        nd_hbm.at[c, pl.ds(sid * NPT, NPT)])
        plsc.subcore_barrier()


def _make_sc(write_ehat):
    mesh = plsc.VectorSubcoreMesh(core_axis_name="c", subcore_axis_name="s")
    out_type = [jax.ShapeDtypeStruct((NCH, NPAD, 2 * CH), F32)]
    if write_ehat:
        out_type += [
            jax.ShapeDtypeStruct((NCH, E, CH), F32),
            jax.ShapeDtypeStruct((NCH, TILES, 8, 16), F32),
        ]
    return functools.partial(
        pl.kernel,
        mesh=mesh,
        out_type=out_type,
        scratch_types=[
            pltpu.VMEM((BE,), jnp.int32),
            pltpu.VMEM((BE,), jnp.int32),
            pltpu.VMEM((BE, 2 * CH), F32),
            pltpu.VMEM((BE, 2 * CH), F32),
            pltpu.VMEM((BE, CH), F32),
            pltpu.VMEM((BE, 2 * CH), F32),
            pltpu.VMEM((8, 16), F32),
            pltpu.VMEM_SHARED((NPAD, 2 * CH), F32),
            pltpu.SemaphoreType.DMA,
            pltpu.SemaphoreType.DMA,
        ],
    )(functools.partial(_sc_body, write_ehat))


def _sc_pass(src, dst, dv, et, ce, write_ehat):
    return _make_sc(write_ehat)(src, dst, dv, et, ce)


def kernel(x, e, edge_index, Wh, We, A, B, C, D, Ew,
           bn_hg, bn_hb, bn_eg, bn_eb, Wout):
    src = edge_index[0].astype(jnp.int32)
    dst = edge_index[1].astype(jnp.int32)

    h = None
    ef = None
    ehat = None
    ae = be = None
    for l in range(NLAYER):
        if l == 0:
            h, uh, dv, et = _node0_mm(x, Wh, A[0], B[0], D[0], Ew[0])
            ef, ce = _edge0_mm(e, We, C[0])
        else:
            uh, dv, et = _node_mm(h, A[l], B[l], D[l], Ew[l])
            ef, ce = _edge_mm(ef, ehat, ae, be, C[l])
        if l < NLAYER - 1:
            nd, ehat, st = _sc_pass(src, dst, dv, et, ce, True)
            s = jnp.sum(st, axis=1)
            s1 = s[:, 0:4, :].reshape(H)
            s2 = s[:, 4:8, :].reshape(H)
            me = s1 / E
            ve = s2 / E - me * me
            rstd = 1.0 / jnp.sqrt(ve + 1e-5)
            ae = (bn_eg[l] * rstd).reshape(1, H)
            be = (bn_eb[l] - me * bn_eg[l] * rstd).reshape(1, H)
        else:
            (nd,) = _sc_pass(src, dst, dv, et, ce, False)
        h = _update(nd, uh, h, bn_hg[l], bn_hb[l])
    return _out_mm(h, Wout)


# speedup vs baseline: 3.2079x; 1.5905x over previous
import functools

import jax
import jax.numpy as jnp
from jax import lax
from jax.experimental import pallas as pl
from jax.experimental.pallas import tpu as pltpu
from jax.experimental.pallas import tpu_sc as plsc

F32 = jnp.float32

H = 256
NLAYER = 4
N = 10000
E = 160000
CH = 64
NCH = 4
TILES = 16
EPT = E // TILES
BE = 40
NBLK = EPT // BE
NPAD = 10112
NPT = NPAD // TILES
NB = 2000
EB = 2000


def _node_body(h_ref, a_ref, b_ref, d_ref, w_ref, uh_ref, dv_ref, et_ref):
    h = h_ref[...]
    uh_ref[...] = jnp.dot(h, a_ref[...], preferred_element_type=F32)
    vh = jnp.dot(h, b_ref[...], preferred_element_type=F32)
    dh = jnp.dot(h, d_ref[...], preferred_element_type=F32)
    ew = jnp.dot(h, w_ref[...], preferred_element_type=F32)
    for c in range(NCH):
        dv_ref[c, :, 0:CH] = dh[:, c * CH:(c + 1) * CH]
        dv_ref[c, :, CH:2 * CH] = vh[:, c * CH:(c + 1) * CH]
    for s in range(2):
        et_ref[s] = ew[:, s * 2 * CH:(s + 1) * 2 * CH]


def _node0_body(x_ref, wh_ref, a_ref, b_ref, d_ref, w_ref,
                h_ref, uh_ref, dv_ref, et_ref):
    h = jnp.dot(x_ref[...], wh_ref[...], preferred_element_type=F32)
    h_ref[...] = h
    uh_ref[...] = jnp.dot(h, a_ref[...], preferred_element_type=F32)
    vh = jnp.dot(h, b_ref[...], preferred_element_type=F32)
    dh = jnp.dot(h, d_ref[...], preferred_element_type=F32)
    ew = jnp.dot(h, w_ref[...], preferred_element_type=F32)
    for c in range(NCH):
        dv_ref[c, :, 0:CH] = dh[:, c * CH:(c + 1) * CH]
        dv_ref[c, :, CH:2 * CH] = vh[:, c * CH:(c + 1) * CH]
    for s in range(2):
        et_ref[s] = ew[:, s * 2 * CH:(s + 1) * 2 * CH]


_W_SPEC = pl.BlockSpec((H, H), lambda i: (0, 0))


def _node_mm(h, a, b, d, w):
    grid = (N // NB,)
    return pl.pallas_call(
        _node_body,
        grid=grid,
        in_specs=[pl.BlockSpec((NB, H), lambda i: (i, 0))] + [_W_SPEC] * 4,
        out_specs=[
            pl.BlockSpec((NB, H), lambda i: (i, 0)),
            pl.BlockSpec((NCH, NB, 2 * CH), lambda i: (0, i, 0)),
            pl.BlockSpec((2, NB, 2 * CH), lambda i: (0, i, 0)),
        ],
        out_shape=[
            jax.ShapeDtypeStruct((N, H), F32),
            jax.ShapeDtypeStruct((NCH, N, 2 * CH), F32),
            jax.ShapeDtypeStruct((2, N, 2 * CH), F32),
        ],
    )(h, a, b, d, w)


def _node0_mm(x, wh, a, b, d, w):
    grid = (N // NB,)
    return pl.pallas_call(
        _node0_body,
        grid=grid,
        in_specs=[pl.BlockSpec((NB, H), lambda i: (i, 0))] + [_W_SPEC] * 5,
        out_specs=[
            pl.BlockSpec((NB, H), lambda i: (i, 0)),
            pl.BlockSpec((NB, H), lambda i: (i, 0)),
            pl.BlockSpec((NCH, NB, 2 * CH), lambda i: (0, i, 0)),
            pl.BlockSpec((2, NB, 2 * CH), lambda i: (0, i, 0)),
        ],
        out_shape=[
            jax.ShapeDtypeStruct((N, H), F32),
            jax.ShapeDtypeStruct((N, H), F32),
            jax.ShapeDtypeStruct((NCH, N, 2 * CH), F32),
            jax.ShapeDtypeStruct((2, N, 2 * CH), F32),
        ],
    )(x, wh, a, b, d, w)


def _edge0_body(e_ref, we_ref, c_ref, ef_ref, ce_ref):
    ef = jnp.dot(e_ref[...], we_ref[...], preferred_element_type=F32)
    ef_ref[...] = ef
    ce = jnp.dot(ef, c_ref[...], preferred_element_type=F32)
    for c in range(NCH):
        ce_ref[c] = ce[:, c * CH:(c + 1) * CH]


def _edge0_mm(e, we, cw):
    grid = (E // EB,)
    de = e.shape[1]
    return pl.pallas_call(
        _edge0_body,
        grid=grid,
        in_specs=[
            pl.BlockSpec((EB, de), lambda i: (i, 0)),
            pl.BlockSpec((de, H), lambda i: (0, 0)),
            _W_SPEC,
        ],
        out_specs=[
            pl.BlockSpec((EB, H), lambda i: (i, 0)),
            pl.BlockSpec((NCH, EB, CH), lambda i: (0, i, 0)),
        ],
        out_shape=[
            jax.ShapeDtypeStruct((E, H), F32),
            jax.ShapeDtypeStruct((NCH, E, CH), F32),
        ],
    )(e, we, cw)


def _edge_body(ef_ref, eh_ref, ae_ref, be_ref, c_ref, efn_ref, ce_ref):
    eh = jnp.concatenate([eh_ref[c] for c in range(NCH)], axis=1)
    ef = ef_ref[...] + jnp.maximum(eh * ae_ref[...] + be_ref[...], 0.0)
    efn_ref[...] = ef
    ce = jnp.dot(ef, c_ref[...], preferred_element_type=F32)
    for c in range(NCH):
        ce_ref[c] = ce[:, c * CH:(c + 1) * CH]


def _edge_mm(ef, ehat, ae, be, cw):
    grid = (E // EB,)
    return pl.pallas_call(
        _edge_body,
        grid=grid,
        in_specs=[
            pl.BlockSpec((EB, H), lambda i: (i, 0)),
            pl.BlockSpec((NCH, EB, CH), lambda i: (0, i, 0)),
            pl.BlockSpec((1, H), lambda i: (0, 0)),
            pl.BlockSpec((1, H), lambda i: (0, 0)),
            _W_SPEC,
        ],
        out_specs=[
            pl.BlockSpec((EB, H), lambda i: (i, 0)),
            pl.BlockSpec((NCH, EB, CH), lambda i: (0, i, 0)),
        ],
        out_shape=[
            jax.ShapeDtypeStruct((E, H), F32),
            jax.ShapeDtypeStruct((NCH, E, CH), F32),
        ],
    )(ef, ehat, ae, be, cw)


def _hhat_body(nd_ref, uh_ref, hh_ref, part_ref):
    num = jnp.concatenate([nd_ref[c, :, 0:CH] for c in range(NCH)], axis=1)
    den = jnp.concatenate([nd_ref[c, :, CH:2 * CH] for c in range(NCH)],
                          axis=1)
    hh = uh_ref[...] + num / (den + 1e-6)
    hh_ref[...] = hh
    part_ref[0, 0] = jnp.sum(hh, axis=0)
    part_ref[0, 1] = jnp.sum(hh * hh, axis=0)


def _hhat(nd, uh):
    grid = (N // NB,)
    return pl.pallas_call(
        _hhat_body,
        grid=grid,
        in_specs=[
            pl.BlockSpec((NCH, NB, 2 * CH), lambda i: (0, i, 0)),
            pl.BlockSpec((NB, H), lambda i: (i, 0)),
        ],
        out_specs=[
            pl.BlockSpec((NB, H), lambda i: (i, 0)),
            pl.BlockSpec((1, 2, H), lambda i: (i, 0, 0)),
        ],
        out_shape=[
            jax.ShapeDtypeStruct((N, H), F32),
            jax.ShapeDtypeStruct((N // NB, 2, H), F32),
        ],
    )(nd, uh)


def _hup_body(hh_ref, h_ref, sa_ref, sb_ref, hn_ref):
    bn = hh_ref[...] * sa_ref[...] + sb_ref[...]
    hn_ref[...] = h_ref[...] + jnp.maximum(bn, 0.0)


def _hup(hh, h, sa, sb):
    grid = (N // NB,)
    return pl.pallas_call(
        _hup_body,
        grid=grid,
        in_specs=[
            pl.BlockSpec((NB, H), lambda i: (i, 0)),
            pl.BlockSpec((NB, H), lambda i: (i, 0)),
            pl.BlockSpec((1, H), lambda i: (0, 0)),
            pl.BlockSpec((1, H), lambda i: (0, 0)),
        ],
        out_specs=pl.BlockSpec((NB, H), lambda i: (i, 0)),
        out_shape=jax.ShapeDtypeStruct((N, H), F32),
    )(hh, h, sa, sb)


def _update(nd, uh, h, g, b):
    hh, part = _hhat(nd, uh)
    m = jnp.sum(part[:, 0, :], axis=0) / N
    ms = jnp.sum(part[:, 1, :], axis=0) / N
    var = ms - m * m
    rstd = 1.0 / jnp.sqrt(var + 1e-5)
    sa = (g * rstd).reshape(1, H)
    sb = (b - m * g * rstd).reshape(1, H)
    return _hup(hh, h, sa, sb)


def _out_body(h_ref, w_ref, o_ref):
    o_ref[...] = jnp.tanh(jnp.dot(h_ref[...], w_ref[...],
                                  preferred_element_type=F32))


def _out_mm(h, wout):
    return pl.pallas_call(
        _out_body,
        in_specs=[
            pl.BlockSpec((N, H), lambda: (0, 0)),
            pl.BlockSpec((H, 8), lambda: (0, 0)),
        ],
        out_specs=pl.BlockSpec((N, 8), lambda: (0, 0)),
        out_shape=jax.ShapeDtypeStruct((N, 8), F32),
    )(h, wout)


def _sc_body(write_ehat, src_hbm, dst_hbm, dv_hbm, et_hbm, ce_hbm, *rest):
    if write_ehat:
        nd_hbm, ehat_hbm, st_hbm = rest[:3]
        rest = rest[3:]
    else:
        nd_hbm = rest[0]
        rest = rest[1:]
    sx = rest[0:4]
    dx = rest[4:8]
    dvb = rest[8:10]
    etb = rest[10:12]
    ceb = rest[12:14]
    scb = rest[14:16]
    st_b = rest[16]
    acc = rest[17]
    six = rest[18:22]
    sdv = rest[22:24]
    sete = rest[24:26]
    sce = rest[26:28]
    ssc = rest[28:30]

    cid = lax.axis_index("c")
    sid = lax.axis_index("s")
    zero = jnp.zeros((16,), F32)

    def base_of(b):
        return sid * EPT + b * BE

    def issue_idx(b, ks):
        pltpu.async_copy(src_hbm.at[pl.ds(base_of(b), BE)], sx[ks], six[ks])
        pltpu.async_copy(dst_hbm.at[pl.ds(base_of(b), BE)], dx[ks], six[ks])

    def wait_idx(ks):
        pltpu.make_async_copy(src_hbm.at[pl.ds(0, BE)], sx[ks],
                              six[ks]).wait()
        pltpu.make_async_copy(dst_hbm.at[pl.ds(0, BE)], dx[ks],
                              six[ks]).wait()

    def issue_gath(b, ks, s, c):
        pltpu.async_copy(dv_hbm.at[c].at[sx[ks]], dvb[s], sdv[s])
        pltpu.async_copy(et_hbm.at[cid].at[dx[ks]], etb[s], sete[s])
        pltpu.async_copy(ce_hbm.at[c, pl.ds(base_of(b), BE)], ceb[s], sce[s])

    def wait_gath(s):
        pltpu.make_async_copy(dv_hbm.at[0, pl.ds(0, BE)], dvb[s],
                              sdv[s]).wait()
        pltpu.make_async_copy(et_hbm.at[0, pl.ds(0, BE)], etb[s],
                              sete[s]).wait()
        pltpu.make_async_copy(ce_hbm.at[0, pl.ds(0, BE)], ceb[s],
                              sce[s]).wait()

    def wait_scat(s):
        pltpu.make_async_copy(scb[s], acc.at[pl.ds(0, BE)], ssc[s]).wait()

    def zrow(r, _):
        for q in range(8):
            scb[0][r, pl.ds(16 * q, 16)] = zero
        return 0

    for j in range(2):
        c = cid * 2 + j
        lax.fori_loop(0, BE, zrow, 0)
        for k in range(15):
            pltpu.sync_copy(scb[0], acc.at[pl.ds(sid * NPT + k * BE, BE)])
        pltpu.sync_copy(scb[0].at[pl.ds(0, NPT - 15 * BE)],
                        acc.at[pl.ds(sid * NPT + 15 * BE, NPT - 15 * BE)])
        plsc.subcore_barrier()

        def compute(b, s, carry, j=j):
            def row(r, rc):
                s1n, s2n = [], []
                for q in range(4):
                    sl = pl.ds(16 * q, 16)
                    sl2 = pl.ds(CH + 16 * q, 16)
                    sle = pl.ds(j * CH + 16 * q, 16)
                    ehv = ceb[s][r, sl] + dvb[s][r, sl] + etb[s][r, sle]
                    sg = 1.0 / (1.0 + jnp.exp(-ehv))
                    if write_ehat:
                        ceb[s][r, sl] = ehv
                        s1n.append(rc[q] + ehv)
                        s2n.append(rc[4 + q] + ehv * ehv)
                    scb[s][r, sl] = sg * dvb[s][r, sl2]
                    scb[s][r, sl2] = sg
                if write_ehat:
                    return tuple(s1n + s2n)
                return rc

            return lax.fori_loop(0, BE, row, carry)

        def process(b, k4, carry, c=c, waits=True, idx_next=True,
                    gath_next=True):
            s = k4 % 2
            wait_gath(s)
            if waits:
                wait_scat(s)
            if idx_next:
                issue_idx(b + 2, (k4 + 2) % 4)
            carry = compute(b, s, carry)
            if write_ehat:
                pltpu.sync_copy(ceb[s], ehat_hbm.at[c, pl.ds(base_of(b), BE)])
            pltpu.async_copy(scb[s], acc.at[dx[k4]], ssc[s], add=True)
            if gath_next:
                wait_idx((k4 + 2) % 4)
                issue_gath(b + 2, (k4 + 2) % 4, s, c)
            return carry

        issue_idx(0, 0)
        issue_idx(1, 1)
        wait_idx(0)
        issue_gath(0, 0, 0, c)
        wait_idx(1)
        issue_gath(1, 1, 1, c)
        issue_idx(2, 2)
        issue_idx(3, 3)

        carry = (zero,) * 8
        carry = process(0, 0, carry, waits=False, idx_next=False)
        carry = process(1, 1, carry, waits=False, idx_next=False)
        carry = process(2, 2, carry)
        carry = process(3, 3, carry)

        def quad(i, carry, c=c):
            b = 4 * i
            for k4 in range(4):
                carry = process(b + k4, k4, carry, c=c)
            return carry

        carry = lax.fori_loop(1, NBLK // 4 - 1, quad, carry)
        b0 = NBLK - 6
        carry = process(b0 + 0, 0, carry)
        carry = process(b0 + 1, 1, carry)
        carry = process(b0 + 2, 2, carry)
        carry = process(b0 + 3, 3, carry)
        carry = process(b0 + 4, 0, carry, idx_next=False, gath_next=False)
        carry = process(b0 + 5, 1, carry, idx_next=False, gath_next=False)
        wait_scat(0)
        wait_scat(1)

        if write_ehat:
            for q in range(4):
                st_b[q] = carry[q]
                st_b[4 + q] = carry[4 + q]
            pltpu.sync_copy(st_b, st_hbm.at[c, sid])
        plsc.subcore_barrier()
        pltpu.sync_copy(acc.at[pl.ds(sid * NPT, NPT)],
                        nd_hbm.at[c, pl.ds(sid * NPT, NPT)])
        plsc.subcore_barrier()


def _make_sc(write_ehat):
    mesh = plsc.VectorSubcoreMesh(core_axis_name="c", subcore_axis_name="s")
    out_type = [jax.ShapeDtypeStruct((NCH, NPAD, 2 * CH), F32)]
    if write_ehat:
        out_type += [
            jax.ShapeDtypeStruct((NCH, E, CH), F32),
            jax.ShapeDtypeStruct((NCH, TILES, 8, 16), F32),
        ]
    return functools.partial(
        pl.kernel,
        mesh=mesh,
        out_type=out_type,
        scratch_types=(
            [pltpu.VMEM((BE,), jnp.int32)] * 8
            + [pltpu.VMEM((BE, 2 * CH), F32)] * 2
            + [pltpu.VMEM((BE, 2 * CH), F32)] * 2
            + [pltpu.VMEM((BE, CH), F32)] * 2
            + [pltpu.VMEM((BE, 2 * CH), F32)] * 2
            + [pltpu.VMEM((8, 16), F32)]
            + [pltpu.VMEM_SHARED((NPAD, 2 * CH), F32)]
            + [pltpu.SemaphoreType.DMA] * 12
        ),
    )(functools.partial(_sc_body, write_ehat))


def _sc_pass(src, dst, dv, et, ce, write_ehat):
    return _make_sc(write_ehat)(src, dst, dv, et, ce)


def kernel(x, e, edge_index, Wh, We, A, B, C, D, Ew,
           bn_hg, bn_hb, bn_eg, bn_eb, Wout):
    src = edge_index[0].astype(jnp.int32)
    dst = edge_index[1].astype(jnp.int32)

    h = None
    ef = None
    ehat = None
    ae = be = None
    for l in range(NLAYER):
        if l == 0:
            h, uh, dv, et = _node0_mm(x, Wh, A[0], B[0], D[0], Ew[0])
            ef, ce = _edge0_mm(e, We, C[0])
        else:
            uh, dv, et = _node_mm(h, A[l], B[l], D[l], Ew[l])
            ef, ce = _edge_mm(ef, ehat, ae, be, C[l])
        if l < NLAYER - 1:
            nd, ehat, st = _sc_pass(src, dst, dv, et, ce, True)
            s = jnp.sum(st, axis=1)
            s1 = s[:, 0:4, :].reshape(H)
            s2 = s[:, 4:8, :].reshape(H)
            me = s1 / E
            ve = s2 / E - me * me
            rstd = 1.0 / jnp.sqrt(ve + 1e-5)
            ae = (bn_eg[l] * rstd).reshape(1, H)
            be = (bn_eb[l] - me * bn_eg[l] * rstd).reshape(1, H)
        else:
            (nd,) = _sc_pass(src, dst, dv, et, ce, False)
        h = _update(nd, uh, h, bn_hg[l], bn_hb[l])
    return _out_mm(h, Wout)


# speedup vs baseline: 3.3865x; 1.0557x over previous
import functools

import jax
import jax.numpy as jnp
import numpy as np
from jax import lax
from jax.experimental import pallas as pl
from jax.experimental.pallas import tpu as pltpu
from jax.experimental.pallas import tpu_sc as plsc

F32 = jnp.float32
BF16 = jnp.bfloat16


H = 256
NLAYER = 4
N = 10000
E = 160000
CH = 64
NCH = 4
TILES = 16
EPT = E // TILES
BE = 40
NBLK = EPT // BE
NPAD = 10112
NPT = NPAD // TILES
NB = 2000
EB = 2000


def _node_body(h_ref, a_ref, b_ref, d_ref, w_ref, uh_ref, dv_ref, et_ref):
    h = h_ref[...]
    uh_ref[...] = jnp.dot(h, a_ref[...], preferred_element_type=F32)
    vh = jnp.dot(h, b_ref[...], preferred_element_type=F32)
    dh = jnp.dot(h, d_ref[...], preferred_element_type=F32)
    ew = jnp.dot(h, w_ref[...], preferred_element_type=F32)
    for c in range(NCH):
        dv_ref[c, :, 0:CH] = dh[:, c * CH:(c + 1) * CH]
        dv_ref[c, :, CH:2 * CH] = vh[:, c * CH:(c + 1) * CH]
    for s in range(2):
        et_ref[s] = ew[:, s * 2 * CH:(s + 1) * 2 * CH]


def _node0_body(x_ref, wh_ref, a_ref, b_ref, d_ref, w_ref,
                h_ref, uh_ref, dv_ref, et_ref):
    h = jnp.dot(x_ref[...], wh_ref[...], preferred_element_type=F32)
    h_ref[...] = h
    uh_ref[...] = jnp.dot(h, a_ref[...], preferred_element_type=F32)
    vh = jnp.dot(h, b_ref[...], preferred_element_type=F32)
    dh = jnp.dot(h, d_ref[...], preferred_element_type=F32)
    ew = jnp.dot(h, w_ref[...], preferred_element_type=F32)
    for c in range(NCH):
        dv_ref[c, :, 0:CH] = dh[:, c * CH:(c + 1) * CH]
        dv_ref[c, :, CH:2 * CH] = vh[:, c * CH:(c + 1) * CH]
    for s in range(2):
        et_ref[s] = ew[:, s * 2 * CH:(s + 1) * 2 * CH]


_W_SPEC = pl.BlockSpec((H, H), lambda i: (0, 0))


def _node_mm(h, a, b, d, w):
    grid = (N // NB,)
    return pl.pallas_call(
        _node_body,
        grid=grid,
        in_specs=[pl.BlockSpec((NB, H), lambda i: (i, 0))] + [_W_SPEC] * 4,
        out_specs=[
            pl.BlockSpec((NB, H), lambda i: (i, 0)),
            pl.BlockSpec((NCH, NB, 2 * CH), lambda i: (0, i, 0)),
            pl.BlockSpec((2, NB, 2 * CH), lambda i: (0, i, 0)),
        ],
        out_shape=[
            jax.ShapeDtypeStruct((N, H), F32),
            jax.ShapeDtypeStruct((NCH, N, 2 * CH), F32),
            jax.ShapeDtypeStruct((2, N, 2 * CH), F32),
        ],
    )(h, a, b, d, w)


def _node0_mm(x, wh, a, b, d, w):
    grid = (N // NB,)
    return pl.pallas_call(
        _node0_body,
        grid=grid,
        in_specs=[pl.BlockSpec((NB, H), lambda i: (i, 0))] + [_W_SPEC] * 5,
        out_specs=[
            pl.BlockSpec((NB, H), lambda i: (i, 0)),
            pl.BlockSpec((NB, H), lambda i: (i, 0)),
            pl.BlockSpec((NCH, NB, 2 * CH), lambda i: (0, i, 0)),
            pl.BlockSpec((2, NB, 2 * CH), lambda i: (0, i, 0)),
        ],
        out_shape=[
            jax.ShapeDtypeStruct((N, H), F32),
            jax.ShapeDtypeStruct((N, H), F32),
            jax.ShapeDtypeStruct((NCH, N, 2 * CH), F32),
            jax.ShapeDtypeStruct((2, N, 2 * CH), F32),
        ],
    )(x, wh, a, b, d, w)


def _edge0_body(e_ref, we_ref, c_ref, ef_ref, ce_ref):
    ef = jnp.dot(e_ref[...], we_ref[...], preferred_element_type=F32)
    ef_ref[...] = ef.astype(BF16)
    ce = jnp.dot(ef, c_ref[...], preferred_element_type=F32)
    for c in range(NCH):
        ce_ref[c] = ce[:, c * CH:(c + 1) * CH]


def _edge0_mm(e, we, cw):
    grid = (E // EB,)
    de = e.shape[1]
    return pl.pallas_call(
        _edge0_body,
        grid=grid,
        in_specs=[
            pl.BlockSpec((EB, de), lambda i: (i, 0)),
            pl.BlockSpec((de, H), lambda i: (0, 0)),
            _W_SPEC,
        ],
        out_specs=[
            pl.BlockSpec((EB, H), lambda i: (i, 0)),
            pl.BlockSpec((NCH, EB, CH), lambda i: (0, i, 0)),
        ],
        out_shape=[
            jax.ShapeDtypeStruct((E, H), BF16),
            jax.ShapeDtypeStruct((NCH, E, CH), F32),
        ],
    )(e, we, cw)


def _edge_body(ef_ref, eh_ref, ae_ref, be_ref, c_ref, efn_ref, ce_ref):
    eh = jnp.concatenate([eh_ref[c] for c in range(NCH)], axis=1)
    ef = (ef_ref[...].astype(F32)
          + jnp.maximum(eh * ae_ref[...] + be_ref[...], 0.0))
    efn_ref[...] = ef.astype(BF16)
    ce = jnp.dot(ef, c_ref[...], preferred_element_type=F32)
    for c in range(NCH):
        ce_ref[c] = ce[:, c * CH:(c + 1) * CH]


def _edge_mm(ef, ehat, ae, be, cw):
    grid = (E // EB,)
    return pl.pallas_call(
        _edge_body,
        grid=grid,
        in_specs=[
            pl.BlockSpec((EB, H), lambda i: (i, 0)),
            pl.BlockSpec((NCH, EB, CH), lambda i: (0, i, 0)),
            pl.BlockSpec((1, H), lambda i: (0, 0)),
            pl.BlockSpec((1, H), lambda i: (0, 0)),
            _W_SPEC,
        ],
        out_specs=[
            pl.BlockSpec((EB, H), lambda i: (i, 0)),
            pl.BlockSpec((NCH, EB, CH), lambda i: (0, i, 0)),
        ],
        out_shape=[
            jax.ShapeDtypeStruct((E, H), BF16),
            jax.ShapeDtypeStruct((NCH, E, CH), F32),
        ],
    )(ef, ehat, ae, be, cw)


def _hhat_body(nd_ref, uh_ref, hh_ref, part_ref):
    num = jnp.concatenate([nd_ref[c, :, 0:CH] for c in range(NCH)], axis=1)
    den = jnp.concatenate([nd_ref[c, :, CH:2 * CH] for c in range(NCH)],
                          axis=1)
    hh = uh_ref[...] + num / (den + 1e-6)
    hh_ref[...] = hh
    part_ref[0, 0] = jnp.sum(hh, axis=0)
    part_ref[0, 1] = jnp.sum(hh * hh, axis=0)


def _hhat(nd, uh):
    grid = (N // NB,)
    return pl.pallas_call(
        _hhat_body,
        grid=grid,
        in_specs=[
            pl.BlockSpec((NCH, NB, 2 * CH), lambda i: (0, i, 0)),
            pl.BlockSpec((NB, H), lambda i: (i, 0)),
        ],
        out_specs=[
            pl.BlockSpec((NB, H), lambda i: (i, 0)),
            pl.BlockSpec((1, 2, H), lambda i: (i, 0, 0)),
        ],
        out_shape=[
            jax.ShapeDtypeStruct((N, H), F32),
            jax.ShapeDtypeStruct((N // NB, 2, H), F32),
        ],
    )(nd, uh)


def _bn_coef(part, g, b):
    m = jnp.sum(part[:, 0, :], axis=0) / N
    ms = jnp.sum(part[:, 1, :], axis=0) / N
    var = ms - m * m
    rstd = 1.0 / jnp.sqrt(var + 1e-5)
    return (g * rstd).reshape(1, H), (b - m * g * rstd).reshape(1, H)


def _nodef_body(hh_ref, hp_ref, sa_ref, sb_ref, a_ref, b_ref, d_ref, w_ref,
                h_ref, uh_ref, dv_ref, et_ref):
    h = hp_ref[...] + jnp.maximum(
        hh_ref[...] * sa_ref[...] + sb_ref[...], 0.0)
    h_ref[...] = h
    uh_ref[...] = jnp.dot(h, a_ref[...], preferred_element_type=F32)
    vh = jnp.dot(h, b_ref[...], preferred_element_type=F32)
    dh = jnp.dot(h, d_ref[...], preferred_element_type=F32)
    ew = jnp.dot(h, w_ref[...], preferred_element_type=F32)
    for c in range(NCH):
        dv_ref[c, :, 0:CH] = dh[:, c * CH:(c + 1) * CH]
        dv_ref[c, :, CH:2 * CH] = vh[:, c * CH:(c + 1) * CH]
    for s in range(2):
        et_ref[s] = ew[:, s * 2 * CH:(s + 1) * 2 * CH]


def _nodef_mm(hh, hp, sa, sb, a, b, d, w):
    grid = (N // NB,)
    return pl.pallas_call(
        _nodef_body,
        grid=grid,
        in_specs=[
            pl.BlockSpec((NB, H), lambda i: (i, 0)),
            pl.BlockSpec((NB, H), lambda i: (i, 0)),
            pl.BlockSpec((1, H), lambda i: (0, 0)),
            pl.BlockSpec((1, H), lambda i: (0, 0)),
        ] + [_W_SPEC] * 4,
        out_specs=[
            pl.BlockSpec((NB, H), lambda i: (i, 0)),
            pl.BlockSpec((NB, H), lambda i: (i, 0)),
            pl.BlockSpec((NCH, NB, 2 * CH), lambda i: (0, i, 0)),
            pl.BlockSpec((2, NB, 2 * CH), lambda i: (0, i, 0)),
        ],
        out_shape=[
            jax.ShapeDtypeStruct((N, H), F32),
            jax.ShapeDtypeStruct((N, H), F32),
            jax.ShapeDtypeStruct((NCH, N, 2 * CH), F32),
            jax.ShapeDtypeStruct((2, N, 2 * CH), F32),
        ],
    )(hh, hp, sa, sb, a, b, d, w)


def _out_body(hh_ref, hp_ref, sa_ref, sb_ref, w_ref, o_ref):
    h = hp_ref[...] + jnp.maximum(
        hh_ref[...] * sa_ref[...] + sb_ref[...], 0.0)
    o_ref[...] = jnp.tanh(jnp.dot(h, w_ref[...],
                                  preferred_element_type=F32))


def _out_mm(hh, hp, sa, sb, wout):
    return pl.pallas_call(
        _out_body,
        in_specs=[
            pl.BlockSpec((N, H), lambda: (0, 0)),
            pl.BlockSpec((N, H), lambda: (0, 0)),
            pl.BlockSpec((1, H), lambda: (0, 0)),
            pl.BlockSpec((1, H), lambda: (0, 0)),
            pl.BlockSpec((H, 8), lambda: (0, 0)),
        ],
        out_specs=pl.BlockSpec((N, 8), lambda: (0, 0)),
        out_shape=jax.ShapeDtypeStruct((N, 8), F32),
    )(hh, hp, sa, sb, wout)


def _sc_body(write_ehat, src_hbm, dst_hbm, dv_hbm, et_hbm, ce_hbm, *rest):
    if write_ehat:
        nd_hbm, ehat_hbm, st_hbm = rest[:3]
        rest = rest[3:]
    else:
        nd_hbm = rest[0]
        rest = rest[1:]
    sx = rest[0:4]
    dx = rest[4:8]
    dvb = rest[8:10]
    etb = rest[10:12]
    ceb = rest[12:14]
    scb = rest[14:16]
    st_b = rest[16]
    acc = rest[17]
    six = rest[18:22]
    sdv = rest[22:24]
    sete = rest[24:26]
    sce = rest[26:28]
    ssc = rest[28:30]

    cid = lax.axis_index("c")
    sid = lax.axis_index("s")
    zero = jnp.zeros((16,), F32)

    def base_of(b):
        return sid * EPT + b * BE

    def issue_idx(b, ks):
        pltpu.async_copy(src_hbm.at[pl.ds(base_of(b), BE)], sx[ks], six[ks])
        pltpu.async_copy(dst_hbm.at[pl.ds(base_of(b), BE)], dx[ks], six[ks])

    def wait_idx(ks):
        pltpu.make_async_copy(src_hbm.at[pl.ds(0, BE)], sx[ks],
                              six[ks]).wait()
        pltpu.make_async_copy(dst_hbm.at[pl.ds(0, BE)], dx[ks],
                              six[ks]).wait()

    def issue_gath(b, ks, s, c):
        pltpu.async_copy(dv_hbm.at[c].at[sx[ks]], dvb[s], sdv[s])
        pltpu.async_copy(et_hbm.at[cid].at[dx[ks]], etb[s], sete[s])
        pltpu.async_copy(ce_hbm.at[c, pl.ds(base_of(b), BE)], ceb[s], sce[s])

    def wait_gath(s):
        pltpu.make_async_copy(dv_hbm.at[0, pl.ds(0, BE)], dvb[s],
                              sdv[s]).wait()
        pltpu.make_async_copy(et_hbm.at[0, pl.ds(0, BE)], etb[s],
                              sete[s]).wait()
        pltpu.make_async_copy(ce_hbm.at[0, pl.ds(0, BE)], ceb[s],
                              sce[s]).wait()

    def wait_scat(s):
        pltpu.make_async_copy(scb[s], acc.at[pl.ds(0, BE)], ssc[s]).wait()

    def zrow(r, _):
        for q in range(8):
            scb[0][r, pl.ds(16 * q, 16)] = zero
        return 0

    for j in range(2):
        c = cid * 2 + j
        lax.fori_loop(0, BE, zrow, 0)
        for k in range(15):
            pltpu.sync_copy(scb[0], acc.at[pl.ds(sid * NPT + k * BE, BE)])
        pltpu.sync_copy(scb[0].at[pl.ds(0, NPT - 15 * BE)],
                        acc.at[pl.ds(sid * NPT + 15 * BE, NPT - 15 * BE)])
        plsc.subcore_barrier()

        def compute(b, s, carry, j=j):
            def row(r, rc):
                s1n, s2n = [], []
                for q in range(4):
                    sl = pl.ds(16 * q, 16)
                    sl2 = pl.ds(CH + 16 * q, 16)
                    sle = pl.ds(j * CH + 16 * q, 16)
                    ehv = ceb[s][r, sl] + dvb[s][r, sl] + etb[s][r, sle]
                    sg = 1.0 / (1.0 + jnp.exp(-ehv))
                    if write_ehat:
                        ceb[s][r, sl] = ehv
                        s1n.append(rc[q] + ehv)
                        s2n.append(rc[4 + q] + ehv * ehv)
                    scb[s][r, sl] = sg * dvb[s][r, sl2]
                    scb[s][r, sl2] = sg
                if write_ehat:
                    return tuple(s1n + s2n)
                return rc

            return lax.fori_loop(0, BE, row, carry)

        def process(b, k4, carry, c=c, waits=True, idx_next=True,
                    gath_next=True):
            s = k4 % 2
            wait_gath(s)
            if waits:
                wait_scat(s)
            if idx_next:
                issue_idx(b + 2, (k4 + 2) % 4)
            carry = compute(b, s, carry)
            if write_ehat:
                pltpu.sync_copy(ceb[s], ehat_hbm.at[c, pl.ds(base_of(b), BE)])
            pltpu.async_copy(scb[s], acc.at[dx[k4]], ssc[s], add=True)
            if gath_next:
                wait_idx((k4 + 2) % 4)
                issue_gath(b + 2, (k4 + 2) % 4, s, c)
            return carry

        issue_idx(0, 0)
        issue_idx(1, 1)
        wait_idx(0)
        issue_gath(0, 0, 0, c)
        wait_idx(1)
        issue_gath(1, 1, 1, c)
        issue_idx(2, 2)
        issue_idx(3, 3)

        carry = (zero,) * 8
        carry = process(0, 0, carry, waits=False, idx_next=False)
        carry = process(1, 1, carry, waits=False, idx_next=False)
        carry = process(2, 2, carry)
        carry = process(3, 3, carry)

        def quad(i, carry, c=c):
            b = 4 * i
            for k4 in range(4):
                carry = process(b + k4, k4, carry, c=c)
            return carry

        carry = lax.fori_loop(1, NBLK // 4 - 1, quad, carry)
        b0 = NBLK - 6
        carry = process(b0 + 0, 0, carry)
        carry = process(b0 + 1, 1, carry)
        carry = process(b0 + 2, 2, carry)
        carry = process(b0 + 3, 3, carry)
        carry = process(b0 + 4, 0, carry, idx_next=False, gath_next=False)
        carry = process(b0 + 5, 1, carry, idx_next=False, gath_next=False)
        wait_scat(0)
        wait_scat(1)

        if write_ehat:
            for q in range(4):
                st_b[q] = carry[q]
                st_b[4 + q] = carry[4 + q]
            pltpu.sync_copy(st_b, st_hbm.at[c, sid])
        plsc.subcore_barrier()
        pltpu.sync_copy(acc.at[pl.ds(sid * NPT, NPT)],
                        nd_hbm.at[c, pl.ds(sid * NPT, NPT)])
        plsc.subcore_barrier()


def _make_sc(write_ehat):
    mesh = plsc.VectorSubcoreMesh(core_axis_name="c", subcore_axis_name="s")
    out_type = [jax.ShapeDtypeStruct((NCH, NPAD, 2 * CH), F32)]
    if write_ehat:
        out_type += [
            jax.ShapeDtypeStruct((NCH, E, CH), F32),
            jax.ShapeDtypeStruct((NCH, TILES, 8, 16), F32),
        ]
    return functools.partial(
        pl.kernel,
        mesh=mesh,
        out_type=out_type,
        scratch_types=(
            [pltpu.VMEM((BE,), jnp.int32)] * 8
            + [pltpu.VMEM((BE, 2 * CH), F32)] * 2
            + [pltpu.VMEM((BE, 2 * CH), F32)] * 2
            + [pltpu.VMEM((BE, CH), F32)] * 2
            + [pltpu.VMEM((BE, 2 * CH), F32)] * 2
            + [pltpu.VMEM((8, 16), F32)]
            + [pltpu.VMEM_SHARED((NPAD, 2 * CH), F32)]
            + [pltpu.SemaphoreType.DMA] * 12
        ),
    )(functools.partial(_sc_body, write_ehat))


def _sc_pass(src, dst, dv, et, ce, write_ehat):
    return _make_sc(write_ehat)(src, dst, dv, et, ce)


def kernel(x, e, edge_index, Wh, We, A, B, C, D, Ew,
           bn_hg, bn_hb, bn_eg, bn_eb, Wout):
    src = edge_index[0].astype(jnp.int32)
    dst = edge_index[1].astype(jnp.int32)
    h = None
    ef = None
    ehat = None
    ae = be = None
    hh = hsa = hsb = None
    for l in range(NLAYER):
        if l == 0:
            h, uh, dv, et = _node0_mm(x, Wh, A[0], B[0], D[0], Ew[0])
            ef, ce = _edge0_mm(e, We, C[0])
        else:
            h, uh, dv, et = _nodef_mm(hh, h, hsa, hsb,
                                      A[l], B[l], D[l], Ew[l])
            ef, ce = _edge_mm(ef, ehat, ae, be, C[l])
        if l < NLAYER - 1:
            nd, ehat, st = _sc_pass(src, dst, dv, et, ce, True)
            s = jnp.sum(st, axis=1)
            s1 = s[:, 0:4, :].reshape(H)
            s2 = s[:, 4:8, :].reshape(H)
            me = s1 / E
            ve = s2 / E - me * me
            rstd = 1.0 / jnp.sqrt(ve + 1e-5)
            ae = (bn_eg[l] * rstd).reshape(1, H)
            be = (bn_eb[l] - me * bn_eg[l] * rstd).reshape(1, H)
        else:
            (nd,) = _sc_pass(src, dst, dv, et, ce, False)
        hh, part = _hhat(nd, uh)
        hsa, hsb = _bn_coef(part, bn_hg[l], bn_hb[l])
    return _out_mm(hh, h, hsa, hsb, Wout)
